# baseline (speedup 1.0000x reference)
import functools
import jax
import jax.numpy as jnp
from jax import lax
from jax.experimental import pallas as pl
from jax.experimental.pallas import tpu as pltpu
from jax.experimental.pallas import tpu_sc as plsc

_N = 10000
_E = 320000
_D = 128
_G = 64
_NC = 2
_NS = 16
_NW = _NC * _NS
_EPW = _E // _NW
_CH = 80
_NCH = 128
_EPAD = _NCH * _CH
_TR = 640
_TR_LAST = _N - (_NS - 1) * _TR
_ACC_ROWS = _NS * _TR
_RB = 400
_NRB = _N // _RB


def _sc_mesh():
    return plsc.VectorSubcoreMesh(
        core_axis_name="c", subcore_axis_name="s",
        num_cores=_NC, num_subcores=_NS)


_DCH = 256
_DN = _EPAD // _DCH


def _deg_kernel_body(dstf_hbm, out_hbm, idx_d, ones_v, acc, sem):
    c = lax.axis_index("c")
    s = lax.axis_index("s")
    wid = s * _NC + c
    pltpu.sync_copy(dstf_hbm.at[wid], idx_d)

    def zrow(r, carry):
        for q in range(_D // 16):
            ones_v[r, pl.ds(q * 16, 16)] = jnp.zeros((16,), jnp.float32)
        return carry
    lax.fori_loop(0, _DCH, zrow, 0)
    base = s * _TR

    @pl.when(s < _NS - 1)
    def _():
        for k in range(_TR // _DCH):
            pltpu.sync_copy(ones_v, acc.at[pl.ds(base + k * _DCH, _DCH)])
        pltpu.sync_copy(ones_v.at[pl.ds(0, _TR % _DCH)],
                        acc.at[pl.ds(base + (_TR // _DCH) * _DCH,
                                     _TR % _DCH)])

    @pl.when(s == _NS - 1)
    def _():
        pltpu.sync_copy(ones_v, acc.at[pl.ds(base, _DCH)])
        pltpu.sync_copy(ones_v.at[pl.ds(0, _TR_LAST - _DCH)],
                        acc.at[pl.ds(base + _DCH, _TR_LAST - _DCH)])

    def orow(r, carry):
        for q in range(_D // 16):
            ones_v[r, pl.ds(q * 16, 16)] = jnp.ones((16,), jnp.float32)
        return carry
    lax.fori_loop(0, _DCH, orow, 0)
    plsc.subcore_barrier()

    depth = 8
    for j in range(depth):
        pltpu.async_copy(ones_v, acc.at[idx_d.at[pl.ds(j * _DCH, _DCH)]],
                         sem, add=True)

    def step(j, carry):
        pltpu.make_async_copy(ones_v, acc.at[pl.ds(0, _DCH)], sem).wait()
        pltpu.async_copy(
            ones_v, acc.at[idx_d.at[pl.ds((j + depth) * _DCH, _DCH)]],
            sem, add=True)
        return carry
    lax.fori_loop(0, _DN - depth, step, 0)
    for j in range(depth):
        pltpu.make_async_copy(ones_v, acc.at[pl.ds(0, _DCH)], sem).wait()

    plsc.subcore_barrier()

    @pl.when(s < _NS - 1)
    def _():
        pltpu.sync_copy(acc.at[pl.ds(base, _TR)],
                        out_hbm.at[c, pl.ds(base, _TR)])

    @pl.when(s == _NS - 1)
    def _():
        pltpu.sync_copy(acc.at[pl.ds(base, _TR_LAST)],
                        out_hbm.at[c, pl.ds(base, _TR_LAST)])


def _deg_call(dst_f):
    f = pl.kernel(
        _deg_kernel_body,
        out_type=jax.ShapeDtypeStruct((_NC, _N, _D), jnp.float32),
        mesh=_sc_mesh(),
        scratch_types=[
            pltpu.VMEM((_EPAD,), jnp.int32),
            pltpu.VMEM((_DCH, _D), jnp.float32),
            pltpu.VMEM_SHARED((_ACC_ROWS, _D), jnp.float32),
            pltpu.SemaphoreType.DMA,
        ],
    )
    return f(dst_f)


_NBUF = 4
_GRP = 32
_NGRP = _NCH // _GRP
_GEDGE = _GRP * _CH


def _layer_kernel_body(val_hbm, srcf_hbm, dstf_hbm, out_hbm,
                       idx_s, idx_d, b0, b1, b2, b3, acc,
                       g0, g1, g2, g3, s0, s1, s2, s3):
    bufs = (b0, b1, b2, b3)
    gsem = (g0, g1, g2, g3)
    ssem = (s0, s1, s2, s3)
    c = lax.axis_index("c")
    s = lax.axis_index("s")
    wid = s * _NC + c

    def zrow(r, carry):
        for q in range(_D // 16):
            b0[r, pl.ds(q * 16, 16)] = jnp.zeros((16,), jnp.float32)
            b1[r, pl.ds(q * 16, 16)] = jnp.zeros((16,), jnp.float32)
        return carry
    lax.fori_loop(0, _CH, zrow, 0)
    base = s * _TR

    @pl.when(s < _NS - 1)
    def _():
        for k in range(_TR // (2 * _CH)):
            pltpu.sync_copy(b0, acc.at[pl.ds(base + 2 * k * _CH, _CH)])
            pltpu.sync_copy(b1, acc.at[pl.ds(base + (2 * k + 1) * _CH, _CH)])

    @pl.when(s == _NS - 1)
    def _():
        for k in range(_TR_LAST // (2 * _CH)):
            pltpu.sync_copy(b0, acc.at[pl.ds(base + 2 * k * _CH, _CH)])
            pltpu.sync_copy(b1, acc.at[pl.ds(base + (2 * k + 1) * _CH, _CH)])

    plsc.subcore_barrier()

    for grp in range(_NGRP):
        pltpu.sync_copy(srcf_hbm.at[wid, pl.ds(grp * _GEDGE, _GEDGE)], idx_s)
        pltpu.sync_copy(dstf_hbm.at[wid, pl.ds(grp * _GEDGE, _GEDGE)], idx_d)
        for b in range(_NBUF):
            pltpu.async_copy(val_hbm.at[idx_s.at[pl.ds(b * _CH, _CH)]],
                             bufs[b], gsem[b])

        def lap(l, carry):
            for b in range(_NBUF):
                off = (l * _NBUF + b) * _CH
                pltpu.make_async_copy(
                    val_hbm.at[pl.ds(0, _CH)], bufs[b], gsem[b]).wait()
                pltpu.async_copy(
                    bufs[b], acc.at[idx_d.at[pl.ds(off, _CH)]],
                    ssem[b], add=True)

                @pl.when(l < _GRP // _NBUF - 1)
                def _():
                    pltpu.make_async_copy(
                        bufs[b], acc.at[pl.ds(0, _CH)], ssem[b]).wait()
                    pltpu.async_copy(
                        val_hbm.at[idx_s.at[pl.ds(off + _NBUF * _CH, _CH)]],
                        bufs[b], gsem[b])
            return carry
        lax.fori_loop(0, _GRP // _NBUF, lap, 0)
        for b in range(_NBUF):
            pltpu.make_async_copy(
                bufs[b], acc.at[pl.ds(0, _CH)], ssem[b]).wait()

    plsc.subcore_barrier()

    @pl.when(s < _NS - 1)
    def _():
        pltpu.sync_copy(acc.at[pl.ds(base, _TR)],
                        out_hbm.at[c, pl.ds(base, _TR)])

    @pl.when(s == _NS - 1)
    def _():
        pltpu.sync_copy(acc.at[pl.ds(base, _TR_LAST)],
                        out_hbm.at[c, pl.ds(base, _TR_LAST)])


def _layer_call(vals, src_f, dst_f):
    f = pl.kernel(
        _layer_kernel_body,
        out_type=jax.ShapeDtypeStruct((_NC, _N, _D), jnp.float32),
        mesh=_sc_mesh(),
        scratch_types=(
            [pltpu.VMEM((_GEDGE,), jnp.int32)] * 2 +
            [pltpu.VMEM((_CH, _D), jnp.float32)] * _NBUF +
            [pltpu.VMEM_SHARED((_ACC_ROWS, _D), jnp.float32)] +
            [pltpu.SemaphoreType.DMA] * (2 * _NBUF)
        ),
    )
    return f(vals, src_f, dst_f)


def _dinv_block(degp):
    deg = degp[0, :, 0] + degp[1, :, 0] + 1.0
    return lax.rsqrt(deg)


def _lrelu(a):
    return jnp.where(a >= 0, a, 0.01 * a)


def _scale_matmul_body(x_ref, w_ref, degp_ref, o_ref, dinv_ref):
    dinv = _dinv_block(degp_ref[...])
    hw = jnp.dot(x_ref[...], w_ref[...], preferred_element_type=jnp.float32)
    o_ref[...] = hw * dinv[:, None]
    dinv_ref[0, 0, :] = dinv


def _scale_matmul(x, w, degp):
    return pl.pallas_call(
        _scale_matmul_body,
        grid=(_NRB,),
        in_specs=[
            pl.BlockSpec((_RB, _D), lambda i: (i, 0)),
            pl.BlockSpec((_D, _D), lambda i: (0, 0)),
            pl.BlockSpec((_NC, _RB, _D), lambda i: (0, i, 0)),
        ],
        out_specs=[
            pl.BlockSpec((_RB, _D), lambda i: (i, 0)),
            pl.BlockSpec((1, 1, _RB), lambda i: (i, 0, 0)),
        ],
        out_shape=[
            jax.ShapeDtypeStruct((_N, _D), jnp.float32),
            jax.ShapeDtypeStruct((_NRB, 1, _RB), jnp.float32),
        ],
    )(x, w, degp)


def _mid_body(acc_ref, hwp_ref, dinv3_ref, b1_ref, w2_ref, o_ref):
    dinv = dinv3_ref[0, 0, :]
    a = acc_ref[...]
    s = a[0] + a[1] + hwp_ref[...]
    h = _lrelu(s * dinv[:, None] + b1_ref[...])
    hw = jnp.dot(h, w2_ref[...], preferred_element_type=jnp.float32)
    o_ref[...] = hw * dinv[:, None]


def _mid_layer(acc, hwp, dinv3, b1, w2):
    return pl.pallas_call(
        _mid_body,
        grid=(_NRB,),
        in_specs=[
            pl.BlockSpec((_NC, _RB, _D), lambda i: (0, i, 0)),
            pl.BlockSpec((_RB, _D), lambda i: (i, 0)),
            pl.BlockSpec((1, 1, _RB), lambda i: (i, 0, 0)),
            pl.BlockSpec((1, _D), lambda i: (0, 0)),
            pl.BlockSpec((_D, _D), lambda i: (0, 0)),
        ],
        out_specs=pl.BlockSpec((_RB, _D), lambda i: (i, 0)),
        out_shape=jax.ShapeDtypeStruct((_N, _D), jnp.float32),
    )(acc, hwp, dinv3, b1, w2)


def _final_body(acc_ref, hwp_ref, dinv3_ref, b2_ref, batch_ref, gf_ref,
                wp_ref, bp_ref, wg_ref, bg_ref, wf1_ref, bf1_ref,
                wf2_ref, bf2_ref, o_ref, psum, cnt):
    i = pl.program_id(0)

    @pl.when(i == 0)
    def _():
        psum[...] = jnp.zeros((_G, _D), jnp.float32)
        cnt[...] = jnp.zeros((_G, _D), jnp.float32)

    dinv = dinv3_ref[0, 0, :]
    a = acc_ref[...]
    s = a[0] + a[1] + hwp_ref[...]
    h2 = _lrelu(s * dinv[:, None] + b2_ref[...])

    b = batch_ref[0, 0, :]
    gids = lax.broadcasted_iota(jnp.int32, (_G, _RB), 0)
    mask = (b[None, :] == gids).astype(jnp.float32)
    psum[...] += jnp.dot(mask, h2, preferred_element_type=jnp.float32)
    cnt[...] += jnp.broadcast_to(
        jnp.sum(mask, axis=1, keepdims=True), (_G, _D))

    @pl.when(i == _NRB - 1)
    def _():
        pooled = psum[...] / jnp.maximum(cnt[...], 1.0)
        gnn = jnp.dot(pooled, wp_ref[...],
                      preferred_element_type=jnp.float32) + bp_ref[...]
        gf = jnp.dot(gf_ref[...], wg_ref[...],
                     preferred_element_type=jnp.float32) + bg_ref[...]
        z = _lrelu(jnp.concatenate([gnn, gf], axis=1))
        z = _lrelu(jnp.dot(z, wf1_ref[...],
                           preferred_element_type=jnp.float32) + bf1_ref[...])
        o_ref[...] = jnp.dot(z, wf2_ref[...],
                             preferred_element_type=jnp.float32) + bf2_ref[...]


def _final(acc, hwp, dinv3, b2, batch3, gf, wp, bp, wg, bg, wf1, bf1, wf2, bf2):
    full = lambda shape: pl.BlockSpec(shape, lambda i: tuple(0 for _ in shape))
    return pl.pallas_call(
        _final_body,
        grid=(_NRB,),
        in_specs=[
            pl.BlockSpec((_NC, _RB, _D), lambda i: (0, i, 0)),
            pl.BlockSpec((_RB, _D), lambda i: (i, 0)),
            pl.BlockSpec((1, 1, _RB), lambda i: (i, 0, 0)),
            full((1, _D)),
            pl.BlockSpec((1, 1, _RB), lambda i: (i, 0, 0)),
            full((_G, _G)),
            full((_D, _D)),
            full((1, _D)),
            full((_G, _D)),
            full((1, _D)),
            full((2 * _D, _D)),
            full((1, _D)),
            full((_D, _D)),
            full((1, _D)),
        ],
        out_specs=pl.BlockSpec((_G, _D), lambda i: (0, 0)),
        out_shape=jax.ShapeDtypeStruct((_G, _D), jnp.float32),
        scratch_shapes=[
            pltpu.VMEM((_G, _D), jnp.float32),
            pltpu.VMEM((_G, _D), jnp.float32),
        ],
    )(acc, hwp, dinv3, b2, batch3, gf, wp, bp, wg, bg, wf1, bf1, wf2, bf2)


def kernel(x, edge_index, batch, graph_feature, W1, b1, W2, b2,
           Wp, bp, Wg, bg, Wf1, bf1, Wf2, bf2):
    src = edge_index[0].reshape(_NW, _EPW)
    dst = edge_index[1].reshape(_NW, _EPW)
    pad = _EPAD - _EPW
    pad_src = (jnp.arange(_NW * pad, dtype=jnp.int32) * 7919) % _N
    pad_dst = _N + (jnp.arange(_NW * pad, dtype=jnp.int32) % (_ACC_ROWS - _N))
    src_f = jnp.concatenate([src, pad_src.reshape(_NW, pad)], axis=1)
    dst_f = jnp.concatenate([dst, pad_dst.astype(jnp.int32).reshape(_NW, pad)],
                            axis=1)

    degp = _deg_call(dst_f)

    b1r = b1.reshape(1, _D)
    b2r = b2.reshape(1, _D)
    batch3 = batch.reshape(_NRB, 1, _RB)

    hw1p, dinv3 = _scale_matmul(x, W1, degp)
    acc1 = _layer_call(hw1p, src_f, dst_f)
    hw2p = _mid_layer(acc1, hw1p, dinv3, b1r, W2)
    acc2 = _layer_call(hw2p, src_f, dst_f)
    return _final(acc2, hw2p, dinv3, b2r, batch3, graph_feature,
                  Wp, bp.reshape(1, _D), Wg, bg.reshape(1, _D),
                  Wf1, bf1.reshape(1, _D), Wf2, bf2.reshape(1, _D))


# speedup vs baseline: 1.0012x; 1.0012x over previous
import functools
import jax
import jax.numpy as jnp
from jax import lax
from jax.experimental import pallas as pl
from jax.experimental.pallas import tpu as pltpu
from jax.experimental.pallas import tpu_sc as plsc

_N = 10000
_E = 320000
_D = 128
_G = 64
_NC = 2
_NS = 16
_NW = _NC * _NS
_EPW = _E // _NW
_CH = 40
_NCH = 256
_EPAD = _NCH * _CH
_TR = 640
_TR_LAST = _N - (_NS - 1) * _TR
_ACC_ROWS = _NS * _TR
_RB = 400
_NRB = _N // _RB


def _sc_mesh():
    return plsc.VectorSubcoreMesh(
        core_axis_name="c", subcore_axis_name="s",
        num_cores=_NC, num_subcores=_NS)


_DCH = 256
_DN = _EPAD // _DCH


def _deg_kernel_body(dstf_hbm, out_hbm, idx_d, ones_v, acc, sem):
    c = lax.axis_index("c")
    s = lax.axis_index("s")
    wid = s * _NC + c
    pltpu.sync_copy(dstf_hbm.at[wid], idx_d)

    def zrow(r, carry):
        for q in range(_D // 16):
            ones_v[r, pl.ds(q * 16, 16)] = jnp.zeros((16,), jnp.float32)
        return carry
    lax.fori_loop(0, _DCH, zrow, 0)
    base = s * _TR

    @pl.when(s < _NS - 1)
    def _():
        for k in range(_TR // _DCH):
            pltpu.sync_copy(ones_v, acc.at[pl.ds(base + k * _DCH, _DCH)])
        pltpu.sync_copy(ones_v.at[pl.ds(0, _TR % _DCH)],
                        acc.at[pl.ds(base + (_TR // _DCH) * _DCH,
                                     _TR % _DCH)])

    @pl.when(s == _NS - 1)
    def _():
        pltpu.sync_copy(ones_v, acc.at[pl.ds(base, _DCH)])
        pltpu.sync_copy(ones_v.at[pl.ds(0, _TR_LAST - _DCH)],
                        acc.at[pl.ds(base + _DCH, _TR_LAST - _DCH)])

    def orow(r, carry):
        for q in range(_D // 16):
            ones_v[r, pl.ds(q * 16, 16)] = jnp.ones((16,), jnp.float32)
        return carry
    lax.fori_loop(0, _DCH, orow, 0)
    plsc.subcore_barrier()

    depth = 8
    for j in range(depth):
        pltpu.async_copy(ones_v, acc.at[idx_d.at[pl.ds(j * _DCH, _DCH)]],
                         sem, add=True)

    def step(j, carry):
        pltpu.make_async_copy(ones_v, acc.at[pl.ds(0, _DCH)], sem).wait()
        pltpu.async_copy(
            ones_v, acc.at[idx_d.at[pl.ds((j + depth) * _DCH, _DCH)]],
            sem, add=True)
        return carry
    lax.fori_loop(0, _DN - depth, step, 0)
    for j in range(depth):
        pltpu.make_async_copy(ones_v, acc.at[pl.ds(0, _DCH)], sem).wait()

    plsc.subcore_barrier()

    @pl.when(s < _NS - 1)
    def _():
        pltpu.sync_copy(acc.at[pl.ds(base, _TR)],
                        out_hbm.at[c, pl.ds(base, _TR)])

    @pl.when(s == _NS - 1)
    def _():
        pltpu.sync_copy(acc.at[pl.ds(base, _TR_LAST)],
                        out_hbm.at[c, pl.ds(base, _TR_LAST)])


def _deg_call(dst_f):
    f = pl.kernel(
        _deg_kernel_body,
        out_type=jax.ShapeDtypeStruct((_NC, _N, _D), jnp.float32),
        mesh=_sc_mesh(),
        scratch_types=[
            pltpu.VMEM((_EPAD,), jnp.int32),
            pltpu.VMEM((_DCH, _D), jnp.float32),
            pltpu.VMEM_SHARED((_ACC_ROWS, _D), jnp.float32),
            pltpu.SemaphoreType.DMA,
        ],
    )
    return f(dst_f)


_NBUF = 8
_GRP = 64
_NGRP = _NCH // _GRP
_GEDGE = _GRP * _CH


def _layer_kernel_body(val_hbm, srcf_hbm, dstf_hbm, out_hbm,
                       idx_s, idx_d, b0, b1, b2, b3, b4, b5, b6, b7, acc,
                       g0, g1, g2, g3, g4, g5, g6, g7,
                       s0, s1, s2, s3, s4, s5, s6, s7):
    bufs = (b0, b1, b2, b3, b4, b5, b6, b7)
    gsem = (g0, g1, g2, g3, g4, g5, g6, g7)
    ssem = (s0, s1, s2, s3, s4, s5, s6, s7)
    c = lax.axis_index("c")
    s = lax.axis_index("s")
    wid = s * _NC + c

    def zrow(r, carry):
        for q in range(_D // 16):
            b0[r, pl.ds(q * 16, 16)] = jnp.zeros((16,), jnp.float32)
            b1[r, pl.ds(q * 16, 16)] = jnp.zeros((16,), jnp.float32)
        return carry
    lax.fori_loop(0, _CH, zrow, 0)
    base = s * _TR

    @pl.when(s < _NS - 1)
    def _():
        for k in range(_TR // (2 * _CH)):
            pltpu.sync_copy(b0, acc.at[pl.ds(base + 2 * k * _CH, _CH)])
            pltpu.sync_copy(b1, acc.at[pl.ds(base + (2 * k + 1) * _CH, _CH)])

    @pl.when(s == _NS - 1)
    def _():
        for k in range(_TR_LAST // (2 * _CH)):
            pltpu.sync_copy(b0, acc.at[pl.ds(base + 2 * k * _CH, _CH)])
            pltpu.sync_copy(b1, acc.at[pl.ds(base + (2 * k + 1) * _CH, _CH)])

    plsc.subcore_barrier()

    for grp in range(_NGRP):
        pltpu.sync_copy(srcf_hbm.at[wid, pl.ds(grp * _GEDGE, _GEDGE)], idx_s)
        pltpu.sync_copy(dstf_hbm.at[wid, pl.ds(grp * _GEDGE, _GEDGE)], idx_d)
        for b in range(_NBUF):
            pltpu.async_copy(val_hbm.at[idx_s.at[pl.ds(b * _CH, _CH)]],
                             bufs[b], gsem[b])

        def lap(l, carry):
            for b in range(_NBUF):
                off = (l * _NBUF + b) * _CH
                pltpu.make_async_copy(
                    val_hbm.at[pl.ds(0, _CH)], bufs[b], gsem[b]).wait()
                pltpu.async_copy(
                    bufs[b], acc.at[idx_d.at[pl.ds(off, _CH)]],
                    ssem[b], add=True)

                @pl.when(l < _GRP // _NBUF - 1)
                def _():
                    pltpu.make_async_copy(
                        bufs[b], acc.at[pl.ds(0, _CH)], ssem[b]).wait()
                    pltpu.async_copy(
                        val_hbm.at[idx_s.at[pl.ds(off + _NBUF * _CH, _CH)]],
                        bufs[b], gsem[b])
            return carry
        lax.fori_loop(0, _GRP // _NBUF, lap, 0)
        for b in range(_NBUF):
            pltpu.make_async_copy(
                bufs[b], acc.at[pl.ds(0, _CH)], ssem[b]).wait()

    plsc.subcore_barrier()

    @pl.when(s < _NS - 1)
    def _():
        pltpu.sync_copy(acc.at[pl.ds(base, _TR)],
                        out_hbm.at[c, pl.ds(base, _TR)])

    @pl.when(s == _NS - 1)
    def _():
        pltpu.sync_copy(acc.at[pl.ds(base, _TR_LAST)],
                        out_hbm.at[c, pl.ds(base, _TR_LAST)])


def _layer_call(vals, src_f, dst_f):
    f = pl.kernel(
        _layer_kernel_body,
        out_type=jax.ShapeDtypeStruct((_NC, _N, _D), jnp.float32),
        mesh=_sc_mesh(),
        scratch_types=(
            [pltpu.VMEM((_GEDGE,), jnp.int32)] * 2 +
            [pltpu.VMEM((_CH, _D), jnp.float32)] * _NBUF +
            [pltpu.VMEM_SHARED((_ACC_ROWS, _D), jnp.float32)] +
            [pltpu.SemaphoreType.DMA] * (2 * _NBUF)
        ),
    )
    return f(vals, src_f, dst_f)


def _dinv_block(degp):
    deg = degp[0, :, 0] + degp[1, :, 0] + 1.0
    return lax.rsqrt(deg)


def _lrelu(a):
    return jnp.where(a >= 0, a, 0.01 * a)


def _scale_matmul_body(x_ref, w_ref, degp_ref, o_ref, dinv_ref):
    dinv = _dinv_block(degp_ref[...])
    hw = jnp.dot(x_ref[...], w_ref[...], preferred_element_type=jnp.float32)
    o_ref[...] = hw * dinv[:, None]
    dinv_ref[0, 0, :] = dinv


def _scale_matmul(x, w, degp):
    return pl.pallas_call(
        _scale_matmul_body,
        grid=(_NRB,),
        in_specs=[
            pl.BlockSpec((_RB, _D), lambda i: (i, 0)),
            pl.BlockSpec((_D, _D), lambda i: (0, 0)),
            pl.BlockSpec((_NC, _RB, _D), lambda i: (0, i, 0)),
        ],
        out_specs=[
            pl.BlockSpec((_RB, _D), lambda i: (i, 0)),
            pl.BlockSpec((1, 1, _RB), lambda i: (i, 0, 0)),
        ],
        out_shape=[
            jax.ShapeDtypeStruct((_N, _D), jnp.float32),
            jax.ShapeDtypeStruct((_NRB, 1, _RB), jnp.float32),
        ],
    )(x, w, degp)


def _mid_body(acc_ref, hwp_ref, dinv3_ref, b1_ref, w2_ref, o_ref):
    dinv = dinv3_ref[0, 0, :]
    a = acc_ref[...]
    s = a[0] + a[1] + hwp_ref[...]
    h = _lrelu(s * dinv[:, None] + b1_ref[...])
    hw = jnp.dot(h, w2_ref[...], preferred_element_type=jnp.float32)
    o_ref[...] = hw * dinv[:, None]


def _mid_layer(acc, hwp, dinv3, b1, w2):
    return pl.pallas_call(
        _mid_body,
        grid=(_NRB,),
        in_specs=[
            pl.BlockSpec((_NC, _RB, _D), lambda i: (0, i, 0)),
            pl.BlockSpec((_RB, _D), lambda i: (i, 0)),
            pl.BlockSpec((1, 1, _RB), lambda i: (i, 0, 0)),
            pl.BlockSpec((1, _D), lambda i: (0, 0)),
            pl.BlockSpec((_D, _D), lambda i: (0, 0)),
        ],
        out_specs=pl.BlockSpec((_RB, _D), lambda i: (i, 0)),
        out_shape=jax.ShapeDtypeStruct((_N, _D), jnp.float32),
    )(acc, hwp, dinv3, b1, w2)


def _final_body(acc_ref, hwp_ref, dinv3_ref, b2_ref, batch_ref, gf_ref,
                wp_ref, bp_ref, wg_ref, bg_ref, wf1_ref, bf1_ref,
                wf2_ref, bf2_ref, o_ref, psum, cnt):
    i = pl.program_id(0)

    @pl.when(i == 0)
    def _():
        psum[...] = jnp.zeros((_G, _D), jnp.float32)
        cnt[...] = jnp.zeros((_G, _D), jnp.float32)

    dinv = dinv3_ref[0, 0, :]
    a = acc_ref[...]
    s = a[0] + a[1] + hwp_ref[...]
    h2 = _lrelu(s * dinv[:, None] + b2_ref[...])

    b = batch_ref[0, 0, :]
    gids = lax.broadcasted_iota(jnp.int32, (_G, _RB), 0)
    mask = (b[None, :] == gids).astype(jnp.float32)
    psum[...] += jnp.dot(mask, h2, preferred_element_type=jnp.float32)
    cnt[...] += jnp.broadcast_to(
        jnp.sum(mask, axis=1, keepdims=True), (_G, _D))

    @pl.when(i == _NRB - 1)
    def _():
        pooled = psum[...] / jnp.maximum(cnt[...], 1.0)
        gnn = jnp.dot(pooled, wp_ref[...],
                      preferred_element_type=jnp.float32) + bp_ref[...]
        gf = jnp.dot(gf_ref[...], wg_ref[...],
                     preferred_element_type=jnp.float32) + bg_ref[...]
        z = _lrelu(jnp.concatenate([gnn, gf], axis=1))
        z = _lrelu(jnp.dot(z, wf1_ref[...],
                           preferred_element_type=jnp.float32) + bf1_ref[...])
        o_ref[...] = jnp.dot(z, wf2_ref[...],
                             preferred_element_type=jnp.float32) + bf2_ref[...]


def _final(acc, hwp, dinv3, b2, batch3, gf, wp, bp, wg, bg, wf1, bf1, wf2, bf2):
    full = lambda shape: pl.BlockSpec(shape, lambda i: tuple(0 for _ in shape))
    return pl.pallas_call(
        _final_body,
        grid=(_NRB,),
        in_specs=[
            pl.BlockSpec((_NC, _RB, _D), lambda i: (0, i, 0)),
            pl.BlockSpec((_RB, _D), lambda i: (i, 0)),
            pl.BlockSpec((1, 1, _RB), lambda i: (i, 0, 0)),
            full((1, _D)),
            pl.BlockSpec((1, 1, _RB), lambda i: (i, 0, 0)),
            full((_G, _G)),
            full((_D, _D)),
            full((1, _D)),
            full((_G, _D)),
            full((1, _D)),
            full((2 * _D, _D)),
            full((1, _D)),
            full((_D, _D)),
            full((1, _D)),
        ],
        out_specs=pl.BlockSpec((_G, _D), lambda i: (0, 0)),
        out_shape=jax.ShapeDtypeStruct((_G, _D), jnp.float32),
        scratch_shapes=[
            pltpu.VMEM((_G, _D), jnp.float32),
            pltpu.VMEM((_G, _D), jnp.float32),
        ],
    )(acc, hwp, dinv3, b2, batch3, gf, wp, bp, wg, bg, wf1, bf1, wf2, bf2)


def kernel(x, edge_index, batch, graph_feature, W1, b1, W2, b2,
           Wp, bp, Wg, bg, Wf1, bf1, Wf2, bf2):
    src = edge_index[0].reshape(_NW, _EPW)
    dst = edge_index[1].reshape(_NW, _EPW)
    pad = _EPAD - _EPW
    pad_src = (jnp.arange(_NW * pad, dtype=jnp.int32) * 7919) % _N
    pad_dst = _N + (jnp.arange(_NW * pad, dtype=jnp.int32) % (_ACC_ROWS - _N))
    src_f = jnp.concatenate([src, pad_src.reshape(_NW, pad)], axis=1)
    dst_f = jnp.concatenate([dst, pad_dst.astype(jnp.int32).reshape(_NW, pad)],
                            axis=1)

    degp = _deg_call(dst_f)

    b1r = b1.reshape(1, _D)
    b2r = b2.reshape(1, _D)
    batch3 = batch.reshape(_NRB, 1, _RB)

    hw1p, dinv3 = _scale_matmul(x, W1, degp)
    acc1 = _layer_call(hw1p, src_f, dst_f)
    hw2p = _mid_layer(acc1, hw1p, dinv3, b1r, W2)
    acc2 = _layer_call(hw2p, src_f, dst_f)
    return _final(acc2, hw2p, dinv3, b2r, batch3, graph_feature,
                  Wp, bp.reshape(1, _D), Wg, bg.reshape(1, _D),
                  Wf1, bf1.reshape(1, _D), Wf2, bf2.reshape(1, _D))
